# Initial kernel scaffold; baseline (speedup 1.0000x reference)
import jax
import jax.numpy as jnp
from jax.experimental import pallas as pl


def kernel(inputs, targets, negatives, emb_u, emb_v):
    raise NotImplementedError("write your pallas kernel here")


# speedup vs baseline: 5.1566x; 5.1566x over previous
import functools

import jax
import jax.numpy as jnp
from jax import lax
from jax.experimental import pallas as pl
from jax.experimental.pallas import tpu as pltpu
from jax.experimental.pallas import tpu_sc as plsc

L = 16


def _sc_partials(V, D, B, N):
    info = plsc.get_sparse_core_info()
    NC, NS = info.num_cores, info.num_subcores
    NW = NC * NS
    BW = B // NW
    C = 32
    NCH = BW // C
    G = D // L
    NEG_PER_GATHER = 128 // N * N
    n_gathers = (C * N + NEG_PER_GATHER - 1) // NEG_PER_GATHER

    mesh = plsc.VectorSubcoreMesh(core_axis_name="c", subcore_axis_name="s")

    @functools.partial(
        pl.kernel,
        out_type=(
            jax.ShapeDtypeStruct((B, L), jnp.float32),
            jax.ShapeDtypeStruct((B, L), jnp.float32),
        ),
        mesh=mesh,
        scratch_types=[
            pltpu.VMEM((C,), jnp.int32),
            pltpu.VMEM((C,), jnp.int32),
            pltpu.VMEM((C * N,), jnp.int32),
            pltpu.VMEM((C, D), jnp.float32),
            pltpu.VMEM((C, D), jnp.float32),
            pltpu.VMEM((C * N, D), jnp.float32),
            pltpu.VMEM((BW, L), jnp.float32),
            pltpu.VMEM((BW, L), jnp.float32),
            pltpu.SemaphoreType.DMA,
            pltpu.SemaphoreType.DMA,
            pltpu.SemaphoreType.DMA,
        ],
        compiler_params=pltpu.CompilerParams(use_tc_tiling_on_sc=False),
    )
    def sc_fn(inputs_hbm, targets_hbm, negflat_hbm, emb_u_hbm, emb_v_hbm,
              pos_out, neg_out,
              idx_u, idx_v, idx_n, u_buf, v_buf, n_buf, pos_acc, neg_acc,
              sem_u, sem_v, sem_n):
        wid = lax.axis_index("s") * NC + lax.axis_index("c")
        base = wid * BW

        def chunk_body(c, _):
            b0 = base + c * C
            pltpu.sync_copy(inputs_hbm.at[pl.ds(b0, C)], idx_u)
            pltpu.sync_copy(targets_hbm.at[pl.ds(b0, C)], idx_v)
            pltpu.sync_copy(negflat_hbm.at[pl.ds(b0 * N, C * N)], idx_n)

            cp_u = pltpu.async_copy(emb_u_hbm.at[idx_u], u_buf, sem_u)
            cp_v = pltpu.async_copy(emb_v_hbm.at[idx_v], v_buf, sem_v)
            cps = []
            off = 0
            for _g in range(n_gathers):
                sz = min(NEG_PER_GATHER, C * N - off)
                cps.append(pltpu.async_copy(
                    emb_v_hbm.at[idx_n.at[pl.ds(off, sz)]],
                    n_buf.at[pl.ds(off, sz)], sem_n))
                off += sz
            cp_u.wait()
            cp_v.wait()
            for cp in cps:
                cp.wait()

            def b_body(i, _):
                row = c * C + i
                pos = jnp.zeros((L,), jnp.float32)
                neg = jnp.zeros((L,), jnp.float32)
                for g in range(G):
                    ug = u_buf[i, pl.ds(g * L, L)]
                    vg = v_buf[i, pl.ds(g * L, L)]
                    pos = pos + ug * vg
                    sg = n_buf[i * N, pl.ds(g * L, L)]
                    for n in range(1, N):
                        sg = sg + n_buf[i * N + n, pl.ds(g * L, L)]
                    neg = neg + ug * sg
                pos_acc[row, :] = pos
                neg_acc[row, :] = neg
                return 0

            lax.fori_loop(0, C, b_body, 0)
            return 0

        lax.fori_loop(0, NCH, chunk_body, 0)
        pltpu.sync_copy(pos_acc, pos_out.at[pl.ds(base, BW)])
        pltpu.sync_copy(neg_acc, neg_out.at[pl.ds(base, BW)])

    return sc_fn


def _tc_loss_body(pos_ref, neg_ref, out_ref):
    pos = jnp.sum(pos_ref[...], axis=1)
    neg = jnp.sum(neg_ref[...], axis=1)
    pls = jax.nn.log_sigmoid(pos)
    nls = jax.nn.log_sigmoid(neg)
    out_ref[0, 0] = -(jnp.mean(pls) - jnp.mean(nls))


def kernel(inputs, targets, negatives, emb_u, emb_v):
    V, D = emb_u.shape
    B = inputs.shape[0]
    N = negatives.shape[1]

    inputs = inputs.astype(jnp.int32)
    targets = targets.astype(jnp.int32)
    negflat = negatives.astype(jnp.int32).reshape(-1)

    sc_fn = _sc_partials(V, D, B, N)
    pos_part, neg_part = sc_fn(inputs, targets, negflat, emb_u, emb_v)

    loss = pl.pallas_call(
        _tc_loss_body,
        out_shape=jax.ShapeDtypeStruct((1, 1), jnp.float32),
        out_specs=pl.BlockSpec(memory_space=pltpu.SMEM),
    )(pos_part, neg_part)
    return loss[0, 0]
